# baseline (speedup 1.0000x reference)
import jax
import jax.numpy as jnp
from jax.experimental import pallas as pl

_BI = 400


def _gcn_kernel(adj_ref, xfull_ref, w_ref, b_ref, out_ref):
    i = pl.program_id(0)
    t = jnp.dot(
        adj_ref[...].astype(jnp.bfloat16),
        xfull_ref[...].astype(jnp.bfloat16),
        preferred_element_type=jnp.float32,
    )
    y = jnp.dot(
        t.astype(jnp.bfloat16),
        w_ref[...].astype(jnp.bfloat16),
        preferred_element_type=jnp.float32,
    ) + b_ref[...]
    y = jnp.where(y >= 0, y, 0.01 * y)
    out_ref[...] = y + xfull_ref[pl.ds(i * _BI, _BI), :]


def kernel(x, adj, W, b):
    n, d = x.shape
    b2 = b.reshape(1, d).astype(jnp.float32)
    out = pl.pallas_call(
        _gcn_kernel,
        grid=(n // _BI,),
        in_specs=[
            pl.BlockSpec((_BI, n), lambda i: (i, 0)),
            pl.BlockSpec((n, d), lambda i: (0, 0)),
            pl.BlockSpec((d, d), lambda i: (0, 0)),
            pl.BlockSpec((1, d), lambda i: (0, 0)),
        ],
        out_specs=pl.BlockSpec((_BI, d), lambda i: (i, 0)),
        out_shape=jax.ShapeDtypeStruct((n, d), jnp.float32),
    )(adj, x, W, b2)
    return out
